# baseline (speedup 1.0000x reference)
import functools

import jax
import jax.numpy as jnp
from jax import lax
from jax.experimental import pallas as pl
from jax.experimental.pallas import tpu as pltpu
from jax.experimental.pallas import tpu_sc as plsc

N_ROWS = 262144
N_CLS = 128
BINS = 10
BLK = 16384
NW = 32
PER_W = N_ROWS // NW
NVEC = PER_W // 16
HIST = BINS * 16

_THRESH = (0.0, 0.05129329487681389, 0.10536051541566849, 0.1625189334154129,
           0.2231435477733612, 0.28768208622932434, 0.3566749691963196,
           0.43078291416168213, 0.5108256340026855, 0.5978370308876038,
           0.6931471824645996)


def _rows_body(x_ref, t_ref, loss_ref):
    x3 = x_ref[...]
    xt = jnp.transpose(x3, (0, 2, 1))
    t3 = t_ref[...][:, None, :]
    rowmax = jnp.max(xt, axis=1, keepdims=True)
    ex = jnp.exp(xt - rowmax)
    ssum = jnp.sum(ex, axis=1, keepdims=True)
    cls = lax.broadcasted_iota(jnp.int32, xt.shape, 1)
    ll = jnp.sum(jnp.where(cls == t3, xt, 0.0), axis=1)
    logz = jnp.log(ssum[:, 0, :]) + rowmax[:, 0, :]
    loss_ref[...] = logz - ll


def _sc_hist_body(loss_hbm, cnt_out, sum_out, loss_v, cnt_v, sum_v):
    wid = lax.axis_index("s") * 2 + lax.axis_index("c")
    base = wid * PER_W
    pltpu.sync_copy(loss_hbm.at[pl.ds(base, PER_W)], loss_v)
    zeros = jnp.zeros((16,), jnp.float32)
    for k in range(BINS):
        cnt_v[pl.ds(k * 16, 16)] = zeros
        sum_v[pl.ds(k * 16, 16)] = zeros
    lane = lax.iota(jnp.int32, 16)
    ones = jnp.ones((16,), jnp.float32)

    def body(i, carry):
        for u in range(4):
            lv = loss_v[pl.ds((i * 4 + u) * 16, 16)]
            c = jnp.zeros((16,), jnp.int32)
            for th in _THRESH:
                c = c + (lv >= th).astype(jnp.int32)
            b = c - 1
            valid = (b >= 0) & (b < BINS)
            idx = jnp.where(valid, b, 0) * 16 + lane
            plsc.addupdate_scatter(cnt_v, [idx], ones, mask=valid)
            plsc.addupdate_scatter(sum_v, [idx], lv, mask=valid)
        return carry

    lax.fori_loop(0, NVEC // 4, body, 0)
    pltpu.sync_copy(cnt_v, cnt_out.at[wid])
    pltpu.sync_copy(sum_v, sum_out.at[wid])


@functools.cache
def _sc_hist():
    return pl.kernel(
        _sc_hist_body,
        out_type=(
            jax.ShapeDtypeStruct((NW, HIST), jnp.float32),
            jax.ShapeDtypeStruct((NW, HIST), jnp.float32),
        ),
        mesh=plsc.VectorSubcoreMesh(core_axis_name="c", subcore_axis_name="s"),
        compiler_params=pltpu.CompilerParams(needs_layout_passes=False),
        scratch_types=[
            pltpu.VMEM((PER_W,), jnp.float32),
            pltpu.VMEM((HIST,), jnp.float32),
            pltpu.VMEM((HIST,), jnp.float32),
        ],
    )


def _combine_body(cnt_ref, sum_ref, out_ref):
    cnt = cnt_ref[...]
    s = sum_ref[...]
    col = lax.broadcasted_iota(jnp.int32, cnt.shape, 1) // 16
    r = jnp.float32(0.0)
    for b in range(BINS):
        m = col == b
        cb = jnp.sum(jnp.where(m, cnt, 0.0))
        sb = jnp.sum(jnp.where(m, s, 0.0))
        r = r + jnp.where(cb > 0.0, 4.0 * sb / jnp.maximum(cb, 1.0), 0.0)
    out_ref[0, 0] = r


def kernel(outputs, targets):
    n, c = outputs.shape
    assert n == N_ROWS and c == N_CLS
    ng = BLK // 128
    loss = pl.pallas_call(
        _rows_body,
        grid=(n // BLK,),
        in_specs=[
            pl.BlockSpec((ng, 128, c), lambda i: (i, 0, 0)),
            pl.BlockSpec((ng, 128), lambda i: (i, 0)),
        ],
        out_specs=pl.BlockSpec((ng, 128), lambda i: (i, 0)),
        out_shape=jax.ShapeDtypeStruct((n // 128, 128), jnp.float32),
    )(outputs.reshape(n // 128, 128, c), targets.reshape(n // 128, 128))

    cnt_p, sum_p = _sc_hist()(loss.reshape(n))

    out = pl.pallas_call(
        _combine_body,
        out_specs=pl.BlockSpec(memory_space=pltpu.SMEM),
        out_shape=jax.ShapeDtypeStruct((1, 1), jnp.float32),
    )(cnt_p, sum_p)
    return out[0, 0]


# speedup vs baseline: 1.0342x; 1.0342x over previous
import functools

import jax
import jax.numpy as jnp
from jax import lax
from jax.experimental import pallas as pl
from jax.experimental.pallas import tpu as pltpu
from jax.experimental.pallas import tpu_sc as plsc

N_ROWS = 262144
N_CLS = 128
BINS = 10
BLK = 32768
NW = 32
PER_W = N_ROWS // NW
NVEC = PER_W // 16
HIST = BINS * 16

_THRESH = (0.0, 0.05129329487681389, 0.10536051541566849, 0.1625189334154129,
           0.2231435477733612, 0.28768208622932434, 0.3566749691963196,
           0.43078291416168213, 0.5108256340026855, 0.5978370308876038,
           0.6931471824645996)


def _rows_body(x_ref, t_ref, loss_ref):
    x3 = x_ref[...]
    xt = jnp.transpose(x3, (0, 2, 1))
    t3 = t_ref[...][:, None, :]
    rowmax = jnp.max(xt, axis=1, keepdims=True)
    ex = jnp.exp(xt - rowmax)
    ssum = jnp.sum(ex, axis=1, keepdims=True)
    cls = lax.broadcasted_iota(jnp.int32, xt.shape, 1)
    ll = jnp.sum(jnp.where(cls == t3, xt, 0.0), axis=1)
    logz = jnp.log(ssum[:, 0, :]) + rowmax[:, 0, :]
    loss_ref[...] = logz - ll


def _sc_hist_body(loss_hbm, cnt_out, sum_out, loss_v, cnt_v, sum_v):
    wid = lax.axis_index("s") * 2 + lax.axis_index("c")
    base = wid * PER_W
    pltpu.sync_copy(loss_hbm.at[pl.ds(base, PER_W)], loss_v)
    zeros = jnp.zeros((16,), jnp.float32)
    for k in range(BINS):
        cnt_v[pl.ds(k * 16, 16)] = zeros
        sum_v[pl.ds(k * 16, 16)] = zeros
    lane = lax.iota(jnp.int32, 16)
    ones = jnp.ones((16,), jnp.float32)

    def body(i, carry):
        for u in range(8):
            lv = loss_v[pl.ds((i * 8 + u) * 16, 16)]
            c = jnp.zeros((16,), jnp.int32)
            for th in _THRESH:
                c = c + (lv >= th).astype(jnp.int32)
            b = c - 1
            valid = (b >= 0) & (b < BINS)
            idx = jnp.where(valid, b, 0) * 16 + lane
            plsc.addupdate_scatter(cnt_v, [idx], ones, mask=valid)
            plsc.addupdate_scatter(sum_v, [idx], lv, mask=valid)
        return carry

    lax.fori_loop(0, NVEC // 8, body, 0)
    pltpu.sync_copy(cnt_v, cnt_out.at[wid])
    pltpu.sync_copy(sum_v, sum_out.at[wid])


@functools.cache
def _sc_hist():
    return pl.kernel(
        _sc_hist_body,
        out_type=(
            jax.ShapeDtypeStruct((NW, HIST), jnp.float32),
            jax.ShapeDtypeStruct((NW, HIST), jnp.float32),
        ),
        mesh=plsc.VectorSubcoreMesh(core_axis_name="c", subcore_axis_name="s"),
        compiler_params=pltpu.CompilerParams(needs_layout_passes=False),
        scratch_types=[
            pltpu.VMEM((PER_W,), jnp.float32),
            pltpu.VMEM((HIST,), jnp.float32),
            pltpu.VMEM((HIST,), jnp.float32),
        ],
    )


def _combine_body(cnt_ref, sum_ref, out_ref):
    cnt = cnt_ref[...]
    s = sum_ref[...]
    col = lax.broadcasted_iota(jnp.int32, cnt.shape, 1) // 16
    r = jnp.float32(0.0)
    for b in range(BINS):
        m = col == b
        cb = jnp.sum(jnp.where(m, cnt, 0.0))
        sb = jnp.sum(jnp.where(m, s, 0.0))
        r = r + jnp.where(cb > 0.0, 4.0 * sb / jnp.maximum(cb, 1.0), 0.0)
    out_ref[0, 0] = r


def kernel(outputs, targets):
    n, c = outputs.shape
    assert n == N_ROWS and c == N_CLS
    ng = BLK // 128
    loss = pl.pallas_call(
        _rows_body,
        grid=(n // BLK,),
        in_specs=[
            pl.BlockSpec((ng, 128, c), lambda i: (i, 0, 0)),
            pl.BlockSpec((ng, 128), lambda i: (i, 0)),
        ],
        out_specs=pl.BlockSpec((ng, 128), lambda i: (i, 0)),
        out_shape=jax.ShapeDtypeStruct((n // 128, 128), jnp.float32),
    )(outputs.reshape(n // 128, 128, c), targets.reshape(n // 128, 128))

    cnt_p, sum_p = _sc_hist()(loss.reshape(n))

    out = pl.pallas_call(
        _combine_body,
        out_specs=pl.BlockSpec(memory_space=pltpu.SMEM),
        out_shape=jax.ShapeDtypeStruct((1, 1), jnp.float32),
    )(cnt_p, sum_p)
    return out[0, 0]
